# baseline (speedup 1.0000x reference)
import functools

import jax
import jax.numpy as jnp
from jax import lax
from jax.experimental import pallas as pl
from jax.experimental.pallas import tpu as pltpu
from jax.experimental.pallas import tpu_sc as plsc

N = 10000
D = 128
F = 16
FA = F + 1
CH = FA * D
E = 80000
EP = 81920
NSUB = 16
EPS = EP // NSUB
C = 8
NCHUNK = EPS // C
CBLK = 64
NBLK = NCHUNK // CBLK
NP = 10240
RPT = NP // NSUB


def _h_matmul(x_st, w_st):
    BN = 400

    def body(x_ref, w_ref, o_ref):
        o_ref[0] = jnp.dot(x_ref[0], w_ref[0], preferred_element_type=jnp.float32)

    return pl.pallas_call(
        body,
        grid=(2, N // BN),
        in_specs=[
            pl.BlockSpec((1, BN, D), lambda t, i: (t, i, 0)),
            pl.BlockSpec((1, D, CH), lambda t, i: (t, 0, 0)),
        ],
        out_specs=pl.BlockSpec((1, BN, CH), lambda t, i: (t, i, 0)),
        out_shape=jax.ShapeDtypeStruct((2, N, CH), jnp.float32),
    )(x_st, w_st)


def _merge(agg_st, xd_st, root_st, bias_st):
    BN = 400

    def body(a_ref, x_ref, r_ref, b_ref, o_ref):
        o_ref[0] = (a_ref[0]
                    + jnp.dot(x_ref[0], r_ref[0], preferred_element_type=jnp.float32)
                    + b_ref[0])

    return pl.pallas_call(
        body,
        grid=(2, N // BN),
        in_specs=[
            pl.BlockSpec((1, BN, D), lambda t, i: (t, i, 0)),
            pl.BlockSpec((1, BN, D), lambda t, i: (t, i, 0)),
            pl.BlockSpec((1, D, D), lambda t, i: (t, 0, 0)),
            pl.BlockSpec((1, 1, D), lambda t, i: (t, 0, 0)),
        ],
        out_specs=pl.BlockSpec((1, BN, D), lambda t, i: (t, i, 0)),
        out_shape=jax.ShapeDtypeStruct((2, N, D), jnp.float32),
    )(agg_st, xd_st, root_st, bias_st)


def _sc_aggregate(h_flat, srcg, dstg, eag, zrows):
    mesh = plsc.VectorSubcoreMesh(core_axis_name="c", subcore_axis_name="s")

    @functools.partial(
        pl.kernel,
        mesh=mesh,
        out_type=jax.ShapeDtypeStruct((2, NP, D), jnp.float32),
        scratch_types=[
            pltpu.VMEM((CBLK, C), jnp.int32),
            pltpu.VMEM((CBLK, C), jnp.int32),
            pltpu.VMEM((C, 32), jnp.float32),
            pltpu.VMEM((C, CH), jnp.float32),
            pltpu.VMEM((C, D), jnp.float32),
            pltpu.VMEM_SHARED((NP, D), jnp.float32),
            pltpu.SemaphoreType.DMA,
        ],
    )
    def k(h_hbm, src_hbm, dst_hbm, ea_hbm, z_hbm, out_hbm,
          sidx_v, didx_v, ea_v, rows_v, msg_v, acc, sem):
        c = lax.axis_index("c")
        s = lax.axis_index("s")

        pltpu.sync_copy(z_hbm, acc.at[pl.ds(s * RPT, RPT)])
        plsc.subcore_barrier()

        def blk_body(b, carry0):
            pltpu.sync_copy(src_hbm.at[c, s, b], sidx_v)
            pltpu.sync_copy(dst_hbm.at[c, s, b], didx_v)

            def chunk_body(g, carry):
                pltpu.sync_copy(ea_hbm.at[c, s, b, g], ea_v)
                pltpu.async_copy(h_hbm.at[sidx_v.at[g]], rows_v, sem).wait()

                def edge_body(e, carry2):
                    ea_lo = ea_v[e, pl.ds(0, 16)]
                    ea_hi = ea_v[e, pl.ds(16, 16)]
                    accs = [jnp.zeros((16,), jnp.float32) for _ in range(8)]
                    for f in range(FA):
                        src_vec = ea_lo if f < 16 else ea_hi
                        lane = jnp.full((16, 1), f % 16, jnp.int32)
                        sp = lax.gather(
                            src_vec, lane,
                            lax.GatherDimensionNumbers(
                                offset_dims=(), collapsed_slice_dims=(0,),
                                start_index_map=(0,)),
                            (1,), mode=lax.GatherScatterMode.PROMISE_IN_BOUNDS)
                        base = f * D
                        for gq in range(8):
                            h = rows_v[e, pl.ds(base + gq * 16, 16)]
                            accs[gq] = accs[gq] + h * sp
                    for gq in range(8):
                        msg_v[e, pl.ds(gq * 16, 16)] = accs[gq]
                    return carry2

                lax.fori_loop(0, C, edge_body, 0, unroll=False)
                pltpu.sync_copy(msg_v, acc.at[didx_v.at[g]], add=True)
                return carry

            lax.fori_loop(0, CBLK, chunk_body, 0, unroll=False)
            return carry0

        lax.fori_loop(0, NBLK, blk_body, 0, unroll=False)
        plsc.subcore_barrier()
        pltpu.sync_copy(acc.at[pl.ds(s * RPT, RPT)],
                        out_hbm.at[c, pl.ds(s * RPT, RPT)])

    return k(h_flat, srcg, dstg, eag, zrows)


def _prep_type(edge_index, edge_attr, t):
    src = jnp.pad(edge_index[0].astype(jnp.int32), (0, EP - E)) + t * N
    dst = jnp.pad(edge_index[1].astype(jnp.int32), (0, EP - E))
    ea = jnp.pad(
        jnp.concatenate([edge_attr, jnp.ones((E, 1), jnp.float32)], axis=1),
        ((0, EP - E), (0, 32 - FA)))
    return (src.reshape(NSUB, NBLK, CBLK, C),
            dst.reshape(NSUB, NBLK, CBLK, C),
            ea.reshape(NSUB, NBLK, CBLK, C, 32))


def _w2(W_msg, b_msg):
    w3 = jnp.concatenate([W_msg.reshape(F, D, D), b_msg.reshape(1, D, D)], axis=0)
    return w3.transpose(1, 0, 2).reshape(D, CH)


def kernel(x_a, x_b, edge_index_ab, edge_attr_ab, edge_index_ba, edge_attr_ba,
           W_msg_ab, b_msg_ab, root_ab, bias_ab,
           W_msg_ba, b_msg_ba, root_ba, bias_ba):
    x_st = jnp.stack([x_b, x_a])
    w_st = jnp.stack([_w2(W_msg_ba, b_msg_ba), _w2(W_msg_ab, b_msg_ab)])
    h_st = _h_matmul(x_st, w_st)
    h_flat = h_st.reshape(2 * N, CH)

    s0, d0, e0 = _prep_type(edge_index_ba, edge_attr_ba, 0)
    s1, d1, e1 = _prep_type(edge_index_ab, edge_attr_ab, 1)
    srcg = jnp.stack([s0, s1])
    dstg = jnp.stack([d0, d1])
    eag = jnp.stack([e0, e1])
    zrows = jnp.zeros((RPT, D), jnp.float32)

    agg = _sc_aggregate(h_flat, srcg, dstg, eag, zrows)

    xd_st = jnp.stack([x_a, x_b])
    root_st = jnp.stack([root_ba, root_ab])
    bias_st = jnp.stack([bias_ba, bias_ab]).reshape(2, 1, D)
    out = _merge(agg, xd_st, root_st, bias_st)
    return (out[0], out[1])
